# baseline (speedup 1.0000x reference)
import functools

import jax
import jax.numpy as jnp
from jax import lax
from jax.experimental import pallas as pl
from jax.experimental.pallas import tpu as pltpu
from jax.experimental.pallas import tpu_sc as plsc

B, F, L, V, D, H = 1024, 33, 20, 100000, 16, 256

NC, NS = 2, 16
NW = NC * NS
PAIRS = B * F
PPW = PAIRS // NW
PPC = 6
CHUNK = PPC * L
NCHUNKS = PPW // PPC
NBUF = 2


def _pool_body(idx_hbm, tables_hbm, out_hbm, idx_v, rows_v, out_v, sem0, sem1):
    wid = lax.axis_index("s") * NC + lax.axis_index("c")
    pltpu.sync_copy(idx_hbm.at[wid], idx_v)
    sems = (sem0, sem1)

    for b in range(NBUF):
        pltpu.async_copy(tables_hbm.at[idx_v.at[b]], rows_v.at[b], sems[b])

    def body(g, carry):
        for b in range(NBUF):
            c = g * NBUF + b
            pltpu.make_async_copy(
                tables_hbm.at[idx_v.at[c]], rows_v.at[b], sems[b]
            ).wait()
            for p in range(PPC):
                acc = rows_v[b, p * L, :]
                for j in range(1, L):
                    acc = acc + rows_v[b, p * L + j, :]
                out_v[c * PPC + p, :] = acc * (1.0 / L)
            nxt = c + NBUF

            @pl.when(nxt < NCHUNKS)
            def _():
                pltpu.async_copy(
                    tables_hbm.at[idx_v.at[nxt]], rows_v.at[b], sems[b]
                )

        return carry

    lax.fori_loop(0, NCHUNKS // NBUF, body, 0)
    pltpu.sync_copy(out_v, out_hbm.at[pl.ds(wid * PPW, PPW)])


@functools.cache
def _pool_kernel():
    return functools.partial(
        pl.kernel,
        out_type=jax.ShapeDtypeStruct((PAIRS, D), jnp.float32),
        mesh=plsc.VectorSubcoreMesh(
            core_axis_name="c", subcore_axis_name="s",
            num_cores=NC, num_subcores=NS,
        ),
        compiler_params=pltpu.CompilerParams(use_tc_tiling_on_sc=False),
        scratch_types=[
            pltpu.VMEM((NCHUNKS, CHUNK), jnp.int32),
            pltpu.VMEM((NBUF, CHUNK, D), jnp.float32),
            pltpu.VMEM((PPW, D), jnp.float32),
            pltpu.SemaphoreType.DMA,
            pltpu.SemaphoreType.DMA,
        ],
    )(_pool_body)


def _mlp_body(dense_ref, pooled_ref, w1d_ref, w1s_ref, b1_ref, w2_ref, b2_ref,
              out_ref):
    h = jnp.dot(pooled_ref[:], w1s_ref[:],
                preferred_element_type=jnp.float32,
                precision=lax.Precision.HIGHEST)
    df = dense_ref[:] * jnp.float32(D)
    for k in range(3):
        h = h + df[:, k:k + 1] * w1d_ref[k:k + 1, :]
    h = jnp.maximum(h + b1_ref[:], 0.0)
    o = jnp.sum(h * w2_ref[:], axis=1, keepdims=True) + b2_ref[:]
    m = jnp.max(o)
    e = jnp.exp(o - m)
    out_ref[:] = e / jnp.sum(e)


def kernel(dense, sparse_idx, tables, W1, b1, W2, b2):
    offs = (jnp.arange(F, dtype=jnp.int32) * V)[None, :, None]
    flat_idx = (sparse_idx.astype(jnp.int32) + offs).reshape(NW, NCHUNKS, CHUNK)
    tables_flat = tables.reshape(F * V, D)

    pooled = _pool_kernel()(flat_idx, tables_flat)
    pooled = pooled.reshape(B, F * D)

    out = pl.pallas_call(
        _mlp_body,
        out_shape=jax.ShapeDtypeStruct((B, 1), jnp.float32),
    )(dense, pooled, W1[:3], W1[3:], b1.reshape(1, H), W2.reshape(1, H),
      b2.reshape(1, 1))
    return out
